# Initial kernel scaffold; baseline (speedup 1.0000x reference)
import jax
import jax.numpy as jnp
from jax.experimental import pallas as pl


def kernel(g, src_emb, hr, weights_zero, weights_first, weights_middle, weights_last, W_cat, b_cat):
    raise NotImplementedError("write your pallas kernel here")


# speedup vs baseline: 1.0021x; 1.0021x over previous
import functools

import jax
import jax.numpy as jnp
from jax import lax
from jax.experimental import pallas as pl
from jax.experimental.pallas import tpu as pltpu

N, E, D = 10000, 160000, 256
RB = 1000
NRB = N // RB
EPS = 1e-5


def _stats_accum(o_ref, step, ts):
    @pl.when(step == 0)
    def _():
        o_ref[...] = jnp.zeros_like(o_ref)
    zero = jnp.zeros((1, D), jnp.float32)
    sums = [jnp.sum(t, axis=0, keepdims=True) for t in ts]
    sqs = [jnp.sum(t * t, axis=0, keepdims=True) for t in ts]
    pad = [zero] * (4 - len(ts))
    o_ref[...] += jnp.concatenate(sums + pad + sqs + pad, axis=0)


def _stats_pre_body(a_ref, b_ref, o_ref):
    a, b = a_ref[...], b_ref[...]
    _stats_accum(o_ref, pl.program_id(0), (a + b, a * b, a - b))


def _stats_mid_body(s_ref, h_ref, o_ref):
    s, h = s_ref[...], h_ref[...]
    _stats_accum(o_ref, pl.program_id(0), (s, s * h, s + h))


def _stats_agg_body(h_ref, s_ref, m_ref, invdeg_ref, o_ref):
    h, s, m = h_ref[...], s_ref[...], m_ref[...]
    mean = s * invdeg_ref[...]
    mx = jnp.where(jnp.isfinite(m), m, 0.0)
    _stats_accum(o_ref, pl.program_id(0), (h, mean, mx, s))


def _row_spec():
    return pl.BlockSpec((RB, D), lambda i: (i, 0))


def _col_spec():
    return pl.BlockSpec((RB, 1), lambda i: (i, 0))


def _stats_out_spec():
    return pl.BlockSpec((8, D), lambda i: (0, 0))


def _stats_pre(a, b):
    return pl.pallas_call(
        _stats_pre_body, grid=(NRB,),
        in_specs=[_row_spec(), _row_spec()],
        out_specs=_stats_out_spec(),
        out_shape=jax.ShapeDtypeStruct((8, D), jnp.float32),
    )(a, b)


def _stats_mid(s, h):
    return pl.pallas_call(
        _stats_mid_body, grid=(NRB,),
        in_specs=[_row_spec(), _row_spec()],
        out_specs=_stats_out_spec(),
        out_shape=jax.ShapeDtypeStruct((8, D), jnp.float32),
    )(s, h)


def _stats_agg(h, s, m, invdeg):
    return pl.pallas_call(
        _stats_agg_body, grid=(NRB,),
        in_specs=[_row_spec(), _row_spec(), _row_spec(), _col_spec()],
        out_specs=_stats_out_spec(),
        out_shape=jax.ShapeDtypeStruct((8, D), jnp.float32),
    )(h, s, m, invdeg)


def _bn_from_stats(t, st, i):
    mu = st[i] * (1.0 / N)
    var = st[4 + i] * (1.0 / N) - mu * mu
    return jax.nn.relu((t - mu) * lax.rsqrt(var + EPS))


def _apply3_body(a_ref, b_ref, st_ref, w_ref, o_ref, *, mid):
    a, b = a_ref[...], b_ref[...]
    st, w = st_ref[...], w_ref[...]
    if mid:
        ts = (a, a * b, a + b)
    else:
        ts = (a + b, a * b, a - b)
    acc = jnp.zeros_like(a)
    for i, t in enumerate(ts):
        acc += w[0, i] * _bn_from_stats(t, st, i)
    o_ref[...] = acc


def _apply3(a, b, st, w, mid):
    return pl.pallas_call(
        functools.partial(_apply3_body, mid=mid), grid=(NRB,),
        in_specs=[_row_spec(), _row_spec(), _stats_out_spec(),
                  pl.BlockSpec((1, 3), lambda i: (0, 0))],
        out_specs=_row_spec(),
        out_shape=jax.ShapeDtypeStruct((N, D), jnp.float32),
    )(a, b, st, w.reshape(1, 3))


def _applyk_body(*refs, k):
    invdeg_ref, w_ref, o_ref = refs[4 * k], refs[4 * k + 1], refs[4 * k + 2]
    invdeg = invdeg_ref[...]
    w = w_ref[...]
    acc = None
    for j in range(k):
        h, s, m = refs[4 * j][...], refs[4 * j + 1][...], refs[4 * j + 2][...]
        st = refs[4 * j + 3][...]
        mean = s * invdeg
        mx = jnp.where(jnp.isfinite(m), m, 0.0)
        c = (w[j, 0] * _bn_from_stats(h, st, 0)
             + w[j, 1] * _bn_from_stats(mean, st, 1)
             + w[j, 2] * _bn_from_stats(mx, st, 2)
             + w[j, 3] * _bn_from_stats(s, st, 3))
        acc = c if acc is None else acc + c
    o_ref[...] = acc


def _applyk(rounds, invdeg, w):
    k = len(rounds)
    in_specs, args = [], []
    for (h, s, m, st) in rounds:
        in_specs += [_row_spec(), _row_spec(), _row_spec(), _stats_out_spec()]
        args += [h, s, m, st]
    in_specs += [_col_spec(), pl.BlockSpec((k, 4), lambda i: (0, 0))]
    args += [invdeg, w]
    return pl.pallas_call(
        functools.partial(_applyk_body, k=k), grid=(NRB,),
        in_specs=in_specs,
        out_specs=_row_spec(),
        out_shape=jax.ShapeDtypeStruct((N, D), jnp.float32),
    )(*args)


def _matmul_body(x1, x2, x3, x4, x5, wt_ref, b_ref, o_ref):
    wt = wt_ref[...]
    acc = b_ref[...]
    for j, x in enumerate((x1, x2, x3, x4, x5)):
        acc = acc + jnp.dot(x[...], wt[j * D:(j + 1) * D, :],
                            preferred_element_type=jnp.float32)
    o_ref[...] = acc


def _final_matmul(states, W_cat, b_cat):
    wt = W_cat.T
    return pl.pallas_call(
        _matmul_body, grid=(NRB,),
        in_specs=[_row_spec()] * 5 + [
            pl.BlockSpec((5 * D, D), lambda i: (0, 0)),
            pl.BlockSpec((1, D), lambda i: (0, 0))],
        out_specs=_row_spec(),
        out_shape=jax.ShapeDtypeStruct((N, D), jnp.float32),
    )(*states, wt, b_cat.reshape(1, D))


def _aggregate(g, h):
    s = jax.ops.segment_sum(h[g[0]], g[1], num_segments=N)
    m = jax.ops.segment_max(h[g[0]], g[1], num_segments=N)
    return s, m


def _degree(g):
    return jax.ops.segment_sum(jnp.ones((E,), jnp.float32), g[1], num_segments=N)


def kernel(g, src_emb, hr, weights_zero, weights_first, weights_middle,
           weights_last, W_cat, b_cat):
    wz, wf, wm, wl = weights_zero, weights_first, weights_middle, weights_last
    deg = _degree(g)
    invdeg = (1.0 / jnp.maximum(deg, 1.0)).reshape(N, 1)

    st0 = _stats_pre(src_emb, hr)
    h_in = _apply3(src_emb, hr, st0, wz[0], mid=False)

    SA, MA = _aggregate(g, h_in)
    stA = _stats_agg(h_in, SA, MA, invdeg)
    rA = (h_in, SA, MA, stA)
    s1 = _applyk([rA], invdeg, wf[0:1])

    SB, MB = _aggregate(g, s1)
    stB = _stats_agg(s1, SB, MB, invdeg)
    rB = (s1, SB, MB, stB)
    s2 = _applyk([rA, rB], invdeg, wf[1:3])

    SC, MC = _aggregate(g, s2)
    stC = _stats_agg(s2, SC, MC, invdeg)
    rC = (s2, SC, MC, stC)
    s3 = _applyk([rA, rB, rC], invdeg, wf[3:6])

    mids = []
    for i, s in enumerate((s1, s2, s3)):
        stm = _stats_mid(s, h_in)
        mids.append(_apply3(s, h_in, stm, wm[i], mid=True))
    m1, m2, m3 = mids

    rDEF = []
    for m in mids:
        S_, M_ = _aggregate(g, m)
        st_ = _stats_agg(m, S_, M_, invdeg)
        rDEF.append((m, S_, M_, st_))
    s4 = _applyk(rDEF, invdeg, wl[0:3])

    SG, MG = _aggregate(g, s4)
    stG = _stats_agg(s4, SG, MG, invdeg)
    rG = (s4, SG, MG, stG)
    s5 = _applyk(rDEF + [rG], invdeg, wl[3:7])

    return _final_matmul([m1, m2, m3, s4, s5], W_cat, b_cat)


# speedup vs baseline: 1.2939x; 1.2913x over previous
import functools

import jax
import jax.numpy as jnp
from jax import lax
from jax.experimental import pallas as pl
from jax.experimental.pallas import tpu as pltpu
from jax.experimental.pallas import tpu_sc as plsc

N, E, D = 10000, 160000, 256
RB = 1000
NRB = N // RB
EPS = 1e-5

NB = 32
BS = 320
NP = NB * BS
DUMP = BS
CORE_ROWS = 16 * BS + 8
CDUMP = 16 * BS
CE = 64
CH = 2000
FLUSH = 2048
FB = 2176
SEG = E + 128
PACKB = 9
PACK = 1 << PACKB
NEG_INF = float("-inf")


def _mesh():
    return plsc.VectorSubcoreMesh(core_axis_name="c", subcore_axis_name="s")


def _prepass_body(g0_hbm, g1_hbm, lotbl, perm, counts,
                  sbuf, dbuf, stg, tmp, lobuf, cbuf, sem):
    c = lax.axis_index("c")
    s = lax.axis_index("s")
    b = c * 16 + s
    lanes = lax.iota(jnp.int32, 16)
    pltpu.sync_copy(lotbl.at[pl.ds(pl.multiple_of(b * 16, 16), 16)], lobuf)
    lov = lobuf[pl.ds(0, 16)]
    tmp[pl.ds(32, 16)] = jnp.zeros((16,), jnp.int32)

    def flush_blocks(committed, nblk):
        def fl(bi, _):
            po = pl.multiple_of(b * SEG + committed + bi * CE, CE)
            pltpu.sync_copy(stg.at[pl.ds(bi * CE, CE)],
                            perm.at[pl.ds(po, CE)])
            return 0

        lax.fori_loop(0, nblk, fl, 0)

    def chunk(ci, carry):
        pltpu.sync_copy(g0_hbm.at[pl.ds(ci * CH, CH)], sbuf)
        pltpu.sync_copy(g1_hbm.at[pl.ds(ci * CH, CH)], dbuf)

        committed, fill = carry

        def vec(vi, fill):
            sl = pl.ds(vi * 16, 16)
            dv = dbuf[sl]
            sv = sbuf[sl]
            dloc = dv - lov
            mv = (dloc >= 0) & (dloc < BS)
            cum = jnp.where(mv, 1, 0)
            for k in (1, 2, 4, 8):
                sh = cum[jnp.maximum(lanes - k, 0)]
                cum = cum + jnp.where(lanes >= k, sh, 0)
            t = lanes + 1
            lo = jnp.minimum(lanes, 0)
            for step in (8, 4, 2, 1):
                probe = jnp.minimum(lo + step, 15)
                cv = cum[probe]
                lo = jnp.where(cv < t, probe, lo)
            c0 = cum[jnp.minimum(lanes, 0)]
            inv = jnp.where(c0 >= t, 0, jnp.minimum(lo + 1, 15))
            vals = sv * PACK + dloc
            stg[pl.ds(fill, 16)] = vals[inv]
            tmp[pl.ds(16, 16)] = cum
            cumld = tmp[pl.ds(16, 16)]
            npop_splat = cumld[jnp.minimum(lanes + 15, 15)]
            tmp[pl.ds(32, 16)] = tmp[pl.ds(32, 16)] + npop_splat
            return fill + cumld[15]

        fill = lax.fori_loop(0, CH // 16, vec, fill)

        nblk = fill // CE
        flush_blocks(committed, nblk)
        base = nblk * CE
        lv = [stg[pl.ds(base + 16 * v, 16)] for v in range(4)]
        for v in range(4):
            stg[pl.ds(16 * v, 16)] = lv[v]
        return committed + nblk * CE, fill - nblk * CE

    committed, fill = lax.fori_loop(0, E // CH, chunk, (0, 0))

    sent = jnp.full((16,), DUMP, jnp.int32)
    for k in range(8):
        stg[pl.ds(fill + 16 * k, 16)] = sent
    flush_blocks(committed, (fill + 128 + CE - 1) // CE)
    cbuf[pl.ds(0, 16)] = tmp[pl.ds(32, 16)]
    pltpu.sync_copy(cbuf, counts.at[pl.ds(pl.multiple_of(b * 16, 16), 16)])


def _sc_prepass(g):
    f = functools.partial(
        pl.kernel, mesh=_mesh(),
        out_type=[jax.ShapeDtypeStruct((NB * SEG,), jnp.int32),
                  jax.ShapeDtypeStruct((NB * 16,), jnp.int32)],
        scratch_types=[
            pltpu.VMEM((CH,), jnp.int32),
            pltpu.VMEM((CH,), jnp.int32),
            pltpu.VMEM((FB,), jnp.int32),
            pltpu.VMEM((48,), jnp.int32),
            pltpu.VMEM((16,), jnp.int32),
            pltpu.VMEM((16,), jnp.int32),
            pltpu.SemaphoreType.DMA,
        ])(_prepass_body)
    lotbl = jnp.repeat(jnp.arange(NB, dtype=jnp.int32) * BS, 16)
    return f(g[0], g[1], lotbl)


def _red_body(h_hbm, perm, counts, out, dout,
              acc, gbuf, pbuf, sidx, dacc, cbuf, sem, *, mode):
    c = lax.axis_index("c")
    s = lax.axis_index("s")
    b = c * 16 + s

    if mode == "sum":
        init16 = jnp.zeros((16,), jnp.float32)
    else:
        init16 = jnp.full((16,), NEG_INF, jnp.float32)
    z16 = jnp.zeros((16,), jnp.float32)

    def init_acc(r, _):
        for j in range(D // 16):
            acc[r, pl.ds(16 * j, 16)] = init16
        dacc[pl.ds(r * 16, 16)] = z16
        return 0

    lax.fori_loop(0, BS + 8, init_acc, 0)

    pltpu.sync_copy(counts.at[pl.ds(pl.multiple_of(b * 16, 16), 16)], cbuf)
    cnt = cbuf[pl.ds(0, 16)][0]
    nchunks = (cnt + CE - 1) // CE
    one16 = jnp.ones((16,), jnp.float32)

    def chunk(ci, _):
        po = pl.multiple_of(b * SEG + ci * CE, CE)
        pltpu.sync_copy(perm.at[pl.ds(po, CE)], pbuf)
        for v in range(CE // 16):
            pv = pbuf[pl.ds(16 * v, 16)]
            sidx[pl.ds(16 * v, 16)] = lax.shift_right_logical(pv, PACKB)
        pltpu.async_copy(h_hbm.at[sidx], gbuf, sem).wait()

        def group(gi, _):
            dl_vec = pbuf[pl.ds(16 * gi, 16)]
            for l in range(16):
                dl = dl_vec[l] & (PACK - 1)
                e = 16 * gi + l
                if mode == "sum":
                    dacc[pl.ds(dl * 16, 16)] = dacc[pl.ds(dl * 16, 16)] + one16
                for j in range(D // 16):
                    sl = pl.ds(j * 16, 16)
                    if mode == "sum":
                        acc[dl, sl] = acc[dl, sl] + gbuf[e, sl]
                    else:
                        acc[dl, sl] = jnp.maximum(acc[dl, sl], gbuf[e, sl])
            return 0

        lax.fori_loop(0, CE // 16, group, 0)
        return 0

    lax.fori_loop(0, nchunks, chunk, 0)

    pltpu.sync_copy(acc.at[pl.ds(0, BS)], out.at[pl.ds(b * BS, BS)])
    pltpu.sync_copy(dacc.at[pl.ds(0, BS * 16)],
                    dout.at[pl.ds(pl.multiple_of(b * BS * 16, 16), BS * 16)])


def _sc_red(h, perm, counts, mode):
    f = functools.partial(
        pl.kernel, mesh=_mesh(),
        out_type=[jax.ShapeDtypeStruct((NP, D), jnp.float32),
                  jax.ShapeDtypeStruct((NP * 16,), jnp.float32)],
        scratch_types=[
            pltpu.VMEM((BS + 8, D), jnp.float32),
            pltpu.VMEM((CE, D), jnp.float32),
            pltpu.VMEM((CE,), jnp.int32),
            pltpu.VMEM((CE,), jnp.int32),
            pltpu.VMEM(((BS + 8) * 16,), jnp.float32),
            pltpu.VMEM((16,), jnp.int32),
            pltpu.SemaphoreType.DMA,
        ])(functools.partial(_red_body, mode=mode))
    return f(h, perm, counts)


def _stats_accum(o_ref, step, ts):
    @pl.when(step == 0)
    def _():
        o_ref[...] = jnp.zeros_like(o_ref)
    zero = jnp.zeros((1, D), jnp.float32)
    sums = [jnp.sum(t, axis=0, keepdims=True) for t in ts]
    sqs = [jnp.sum(t * t, axis=0, keepdims=True) for t in ts]
    pad = [zero] * (4 - len(ts))
    o_ref[...] += jnp.concatenate(sums + pad + sqs + pad, axis=0)


def _stats_pre_body(a_ref, b_ref, o_ref):
    a, b = a_ref[...], b_ref[...]
    _stats_accum(o_ref, pl.program_id(0), (a + b, a * b, a - b))


def _stats_mid_body(s_ref, h_ref, o_ref):
    s, h = s_ref[...], h_ref[...]
    _stats_accum(o_ref, pl.program_id(0), (s, s * h, s + h))


def _stats_agg_body(h_ref, s_ref, m_ref, deg_ref, o_ref):
    h, s, m = h_ref[...], s_ref[...], m_ref[...]
    mean = s * (1.0 / jnp.maximum(deg_ref[...], 1.0))
    mx = jnp.where(jnp.isfinite(m), m, 0.0)
    _stats_accum(o_ref, pl.program_id(0), (h, mean, mx, s))


def _row_spec():
    return pl.BlockSpec((RB, D), lambda i: (i, 0))


def _col_spec():
    return pl.BlockSpec((RB, 1), lambda i: (i, 0))


def _stats_out_spec():
    return pl.BlockSpec((8, D), lambda i: (0, 0))


def _stats_pre(a, b):
    return pl.pallas_call(
        _stats_pre_body, grid=(NRB,),
        in_specs=[_row_spec(), _row_spec()],
        out_specs=_stats_out_spec(),
        out_shape=jax.ShapeDtypeStruct((8, D), jnp.float32),
    )(a, b)


def _stats_mid(s, h):
    return pl.pallas_call(
        _stats_mid_body, grid=(NRB,),
        in_specs=[_row_spec(), _row_spec()],
        out_specs=_stats_out_spec(),
        out_shape=jax.ShapeDtypeStruct((8, D), jnp.float32),
    )(s, h)


def _stats_agg(h, s, m, deg):
    return pl.pallas_call(
        _stats_agg_body, grid=(NRB,),
        in_specs=[_row_spec(), _row_spec(), _row_spec(), _col_spec()],
        out_specs=_stats_out_spec(),
        out_shape=jax.ShapeDtypeStruct((8, D), jnp.float32),
    )(h, s, m, deg)


def _bn_from_stats(t, st, i):
    mu = st[i] * (1.0 / N)
    var = st[4 + i] * (1.0 / N) - mu * mu
    return jax.nn.relu((t - mu) * lax.rsqrt(var + EPS))


def _apply3_body(a_ref, b_ref, st_ref, w_ref, o_ref, *, mid):
    a, b = a_ref[...], b_ref[...]
    st, w = st_ref[...], w_ref[...]
    if mid:
        ts = (a, a * b, a + b)
    else:
        ts = (a + b, a * b, a - b)
    acc = jnp.zeros_like(a)
    for i, t in enumerate(ts):
        acc += w[0, i] * _bn_from_stats(t, st, i)
    o_ref[...] = acc


def _apply3(a, b, st, w, mid):
    return pl.pallas_call(
        functools.partial(_apply3_body, mid=mid), grid=(NRB,),
        in_specs=[_row_spec(), _row_spec(), _stats_out_spec(),
                  pl.BlockSpec((1, 3), lambda i: (0, 0))],
        out_specs=_row_spec(),
        out_shape=jax.ShapeDtypeStruct((N, D), jnp.float32),
    )(a, b, st, w.reshape(1, 3))


def _applyk_body(*refs, k):
    deg_ref, w_ref, o_ref = refs[4 * k], refs[4 * k + 1], refs[4 * k + 2]
    invdeg = 1.0 / jnp.maximum(deg_ref[...], 1.0)
    w = w_ref[...]
    acc = None
    for j in range(k):
        h, s, m = refs[4 * j][...], refs[4 * j + 1][...], refs[4 * j + 2][...]
        st = refs[4 * j + 3][...]
        mean = s * invdeg
        mx = jnp.where(jnp.isfinite(m), m, 0.0)
        cb = (w[j, 0] * _bn_from_stats(h, st, 0)
              + w[j, 1] * _bn_from_stats(mean, st, 1)
              + w[j, 2] * _bn_from_stats(mx, st, 2)
              + w[j, 3] * _bn_from_stats(s, st, 3))
        acc = cb if acc is None else acc + cb
    o_ref[...] = acc


def _applyk(rounds, deg, w):
    k = len(rounds)
    in_specs, args = [], []
    for (h, s, m, st) in rounds:
        in_specs += [_row_spec(), _row_spec(), _row_spec(), _stats_out_spec()]
        args += [h, s, m, st]
    in_specs += [_col_spec(), pl.BlockSpec((k, 4), lambda i: (0, 0))]
    args += [deg, w]
    return pl.pallas_call(
        functools.partial(_applyk_body, k=k), grid=(NRB,),
        in_specs=in_specs,
        out_specs=_row_spec(),
        out_shape=jax.ShapeDtypeStruct((N, D), jnp.float32),
    )(*args)


def _matmul_body(x1, x2, x3, x4, x5, wt_ref, b_ref, o_ref):
    wt = wt_ref[...]
    acc = b_ref[...]
    for j, x in enumerate((x1, x2, x3, x4, x5)):
        acc = acc + jnp.dot(x[...], wt[j * D:(j + 1) * D, :],
                            preferred_element_type=jnp.float32)
    o_ref[...] = acc


def _final_matmul(states, W_cat, b_cat):
    wt = W_cat.T
    return pl.pallas_call(
        _matmul_body, grid=(NRB,),
        in_specs=[_row_spec()] * 5 + [
            pl.BlockSpec((5 * D, D), lambda i: (0, 0)),
            pl.BlockSpec((1, D), lambda i: (0, 0))],
        out_specs=_row_spec(),
        out_shape=jax.ShapeDtypeStruct((N, D), jnp.float32),
    )(*states, wt, b_cat.reshape(1, D))


def kernel(g, src_emb, hr, weights_zero, weights_first, weights_middle,
           weights_last, W_cat, b_cat):
    wz, wf, wm, wl = weights_zero, weights_first, weights_middle, weights_last
    perm, counts = _sc_prepass(g)

    def aggregate(h):
        su, dg = _sc_red(h, perm, counts, "sum")
        mx, _ = _sc_red(h, perm, counts, "max")
        return su[:N], mx[:N], dg.reshape(NP, 16)

    st0 = _stats_pre(src_emb, hr)
    h_in = _apply3(src_emb, hr, st0, wz[0], mid=False)

    SA, MA, degw = aggregate(h_in)
    deg = degw[:N, 0:1]
    stA = _stats_agg(h_in, SA, MA, deg)
    rA = (h_in, SA, MA, stA)
    s1 = _applyk([rA], deg, wf[0:1])

    SB, MB, _ = aggregate(s1)
    stB = _stats_agg(s1, SB, MB, deg)
    rB = (s1, SB, MB, stB)
    s2 = _applyk([rA, rB], deg, wf[1:3])

    SC, MC, _ = aggregate(s2)
    stC = _stats_agg(s2, SC, MC, deg)
    rC = (s2, SC, MC, stC)
    s3 = _applyk([rA, rB, rC], deg, wf[3:6])

    mids = []
    for i, sv in enumerate((s1, s2, s3)):
        stm = _stats_mid(sv, h_in)
        mids.append(_apply3(sv, h_in, stm, wm[i], mid=True))
    m1, m2, m3 = mids

    rDEF = []
    for m in mids:
        S_, M_, _ = aggregate(m)
        st_ = _stats_agg(m, S_, M_, deg)
        rDEF.append((m, S_, M_, st_))
    s4 = _applyk(rDEF, deg, wl[0:3])

    SG, MG, _ = aggregate(s4)
    stG = _stats_agg(s4, SG, MG, deg)
    rG = (s4, SG, MG, stG)
    s5 = _applyk(rDEF + [rG], deg, wl[3:7])

    return _final_matmul([m1, m2, m3, s4, s5], W_cat, b_cat)
